# Initial kernel scaffold; baseline (speedup 1.0000x reference)
import jax
import jax.numpy as jnp
from jax.experimental import pallas as pl


def kernel(x, gamma, beta):
    raise NotImplementedError("write your pallas kernel here")


# speedup vs baseline: 1.0501x; 1.0501x over previous
import functools

import jax
import jax.numpy as jnp
from jax.experimental import pallas as pl
from jax.experimental.pallas import tpu as pltpu

_EPS = 1e-5


def _fused_kernel(inv_count, x_ref, g_ref, b_ref, o_ref):
    x1 = x_ref[:, 0:1].astype(jnp.float32)
    x2 = x_ref[:, 1:2].astype(jnp.float32)
    ymin = jnp.minimum(x1, x2)
    ymax = jnp.maximum(x1, x2)

    smin = jnp.sum(ymin, axis=(0, 3), keepdims=True)
    smax = jnp.sum(ymax, axis=(0, 3), keepdims=True)
    qmin = jnp.sum(ymin * ymin, axis=(0, 3), keepdims=True)
    qmax = jnp.sum(ymax * ymax, axis=(0, 3), keepdims=True)

    mean_min = smin * inv_count
    mean_max = smax * inv_count
    var_min = jnp.maximum(qmin * inv_count - mean_min * mean_min, 0.0)
    var_max = jnp.maximum(qmax * inv_count - mean_max * mean_max, 0.0)

    g_min = g_ref[0:1, :, :][None]
    g_max = g_ref[1:2, :, :][None]
    b_min = b_ref[0:1, :, :][None]
    b_max = b_ref[1:2, :, :][None]
    scale_min = g_min * jax.lax.rsqrt(var_min + _EPS)
    scale_max = g_max * jax.lax.rsqrt(var_max + _EPS)
    shift_min = b_min - mean_min * scale_min
    shift_max = b_max - mean_max * scale_max

    o_ref[:, 0:1] = (ymin * scale_min + shift_min).astype(o_ref.dtype)
    o_ref[:, 1:2] = (ymax * scale_max + shift_max).astype(o_ref.dtype)


def _pick_c_tile(C):
    for c_t in (8, 16, 4, 2, 1):
        if C % c_t == 0:
            return c_t
    return C


@jax.jit
def _min_max_bn(x, gamma, beta):
    N, C2, H, W = x.shape
    C = C2 // 2
    HW = H * W

    xr = x.reshape(N, 2, C, HW)
    c_t = _pick_c_tile(C)
    grid = (C // c_t,)

    x_spec = pl.BlockSpec((N, 2, c_t, HW), lambda c: (0, 0, c, 0))
    p_spec = pl.BlockSpec((2, c_t, 1), lambda c: (0, c, 0))

    out = pl.pallas_call(
        functools.partial(_fused_kernel, 1.0 / float(N * HW)),
        out_shape=jax.ShapeDtypeStruct((N, 2, C, HW), x.dtype),
        grid=grid,
        in_specs=[x_spec, p_spec, p_spec],
        out_specs=x_spec,
        compiler_params=pltpu.CompilerParams(
            dimension_semantics=("parallel",),
            vmem_limit_bytes=48 << 20),
    )(xr,
      gamma.astype(jnp.float32).reshape(2, C, 1),
      beta.astype(jnp.float32).reshape(2, C, 1))

    return out.reshape(N, -1)


def kernel(x, gamma, beta):
    return _min_max_bn(x, gamma, beta)
